# baseline (speedup 1.0000x reference)
import functools

import jax
import jax.numpy as jnp
from jax import lax
from jax.experimental import pallas as pl
from jax.experimental.pallas import tpu as pltpu
from jax.experimental.pallas import tpu_sc as plsc

PAD = 128
LANES = 16


@functools.lru_cache(maxsize=None)
def _make_lookup(batch, seq, embed, nc, ns):
    nw = nc * ns
    bpw = batch // nw
    assert batch % nw == 0 and bpw == 128 and embed % 8 == 0
    mesh = plsc.VectorSubcoreMesh(core_axis_name="c", subcore_axis_name="s")

    @functools.partial(
        pl.kernel,
        out_type=jax.ShapeDtypeStruct((seq, embed, batch), jnp.float32),
        mesh=mesh,
        scratch_types=[
            pltpu.VMEM((bpw // 8, seq), jnp.int32),
            pltpu.VMEM((seq, bpw), jnp.int32),
            pltpu.VMEM((4, bpw, PAD), jnp.float32),
            pltpu.VMEM((2, embed, bpw), jnp.float32),
            pltpu.SemaphoreType.DMA,
            pltpu.SemaphoreType.DMA,
            pltpu.SemaphoreType.DMA,
            pltpu.SemaphoreType.DMA,
            pltpu.SemaphoreType.DMA,
            pltpu.SemaphoreType.DMA,
        ],
        compiler_params=pltpu.CompilerParams(needs_layout_passes=False),
    )
    def lookup(idx_hbm, table_hbm, out_hbm, idx_v, idxt_v, rows_v, outt_v,
               sg0, sg1, sg2, sg3, sw0, sw1):
        wid = lax.axis_index("s") * nc + lax.axis_index("c")
        b_base = wid * bpw
        sg = (sg0, sg1, sg2, sg3)
        sw = (sw0, sw1)

        lane = lax.iota(jnp.int32, LANES)
        bq = bpw // 8
        for c in range(8):
            pltpu.sync_copy(idx_hbm.at[pl.ds(b_base + c * bq, bq)], idx_v)

            def idxt_step(s, _, c=c):
                s_vec = jnp.full((LANES,), s, jnp.int32)
                vs = []
                for k in range(bq // LANES):
                    b = k * LANES + lane
                    vs.append((c * bq + b, plsc.load_gather(idx_v, [b, s_vec])))
                for bout, v in vs:
                    plsc.store_scatter(idxt_v, [s_vec, bout], v)
                return 0

            lax.fori_loop(0, seq, idxt_step, 0)

        def fire_gather(s, p):
            return pltpu.async_copy(
                table_hbm.at[idxt_v.at[s]], rows_v.at[p], sg[p]
            )

        def transpose_block(p3, p2):
            for e in range(embed):
                e_vec = jnp.full((LANES,), e, jnp.int32)
                vs = []
                for k in range(bpw // LANES):
                    b = k * LANES + lane
                    vs.append(
                        (b, plsc.load_gather(rows_v.at[p3], [b, e_vec]))
                    )
                for b, v in vs:
                    plsc.store_scatter(outt_v.at[p2], [e_vec, b], v)

        def fire_write(s, p):
            return pltpu.async_copy(
                outt_v.at[p], out_hbm.at[s, :, pl.ds(b_base, bpw)], sw[p]
            )

        for s0 in range(3):
            fire_gather(s0, s0)

        def superstep(t, _):
            for p4 in range(4):
                s = 4 * t + p4
                p3 = p4
                p2 = p4 % 2

                @pl.when(s + 3 < seq)
                def _():
                    fire_gather(s + 3, (p4 + 3) % 4)

                pltpu.make_async_copy(
                    table_hbm.at[idxt_v.at[s]], rows_v.at[p3], sg[p3]
                ).wait()
                @pl.when(s >= 2)
                def _():
                    pltpu.make_async_copy(
                        outt_v.at[p2], out_hbm.at[0, :, pl.ds(b_base, bpw)],
                        sw[p2],
                    ).wait()

                transpose_block(p3, p2)
                fire_write(s, p2)
            return 0

        lax.fori_loop(0, seq // 4, superstep, 0)

        for p2 in range(2):
            pltpu.make_async_copy(
                outt_v.at[p2], out_hbm.at[0, :, pl.ds(b_base, bpw)], sw[p2]
            ).wait()

    return lookup


def kernel(inputs, embedding_table):
    b, s = inputs.shape
    v, e = embedding_table.shape
    table_p = jnp.pad(embedding_table, ((0, 0), (0, PAD - e)))
    info = plsc.get_sparse_core_info()
    lookup = _make_lookup(b, s, e, info.num_cores, info.num_subcores)
    out_t = lookup(inputs, table_p)
    return jnp.transpose(out_t, (2, 0, 1))


# speedup vs baseline: 1.7057x; 1.7057x over previous
import functools

import jax
import jax.numpy as jnp
from jax import lax
from jax.experimental import pallas as pl
from jax.experimental.pallas import tpu as pltpu
from jax.experimental.pallas import tpu_sc as plsc


@functools.lru_cache(maxsize=None)
def _make_lookup(batch, seq, embed, nc, ns, nb):
    nw = nc * ns
    rows_per_w = batch // nw
    n_chunks = rows_per_w // nb
    assert batch % nw == 0 and rows_per_w % nb == 0
    assert n_chunks >= 4 and n_chunks % 2 == 0
    splits = []
    off = 0
    while off < seq:
        width = min(128, seq - off)
        splits.append((off, width))
        off += width
    mesh = plsc.VectorSubcoreMesh(core_axis_name="c", subcore_axis_name="s")

    @functools.partial(
        pl.kernel,
        out_type=jax.ShapeDtypeStruct((batch, seq, embed), jnp.float32),
        mesh=mesh,
        scratch_types=[
            pltpu.VMEM((2, nb, seq), jnp.int32),
            pltpu.VMEM((2, nb, seq, embed), jnp.float32),
            pltpu.SemaphoreType.DMA,
            pltpu.SemaphoreType.DMA,
            pltpu.SemaphoreType.DMA,
            pltpu.SemaphoreType.DMA,
        ],
        compiler_params=pltpu.CompilerParams(use_tc_tiling_on_sc=False),
    )
    def lookup(idx_hbm, table_hbm, out_hbm, idx_v, rows_v, sg0, sg1, sw0, sw1):
        wid = lax.axis_index("s") * nc + lax.axis_index("c")
        b_base = wid * rows_per_w
        sg = (sg0, sg1)
        sw = (sw0, sw1)

        def load_idx(c, p):
            pltpu.sync_copy(idx_hbm.at[pl.ds(b_base + c * nb, nb)], idx_v.at[p])

        def fire_gathers(p):
            for j in range(nb):
                for off, width in splits:
                    pltpu.async_copy(
                        table_hbm.at[idx_v.at[p].at[j, pl.ds(off, width)]],
                        rows_v.at[p].at[j, pl.ds(off, width)],
                        sg[p],
                    )

        def drain_gathers(p):
            pltpu.make_async_copy(
                out_hbm.at[pl.ds(0, nb)], rows_v.at[p], sg[p]
            ).wait()

        def fire_write(c, p):
            return pltpu.async_copy(
                rows_v.at[p], out_hbm.at[pl.ds(b_base + c * nb, nb)], sw[p]
            )

        load_idx(0, 0)
        fire_gathers(0)
        load_idx(1, 1)
        fire_gathers(1)

        def superstep(s, _):
            for p in range(2):
                c = 2 * s + p
                drain_gathers(p)
                w = fire_write(c, p)
                load_idx(c + 2, p)
                w.wait()
                fire_gathers(p)
            return 0

        lax.fori_loop(0, (n_chunks - 2) // 2, superstep, 0)

        for p in range(2):
            c = n_chunks - 2 + p
            drain_gathers(p)
            fire_write(c, p).wait()

    return lookup


def kernel(inputs, embedding_table):
    b, s = inputs.shape
    _, e = embedding_table.shape
    info = plsc.get_sparse_core_info()
    lookup = _make_lookup(b, s, e, info.num_cores, info.num_subcores, 4)
    return lookup(inputs, embedding_table)
